# baseline (speedup 1.0000x reference)
import functools

import jax
import jax.numpy as jnp
from jax import lax
from jax.experimental import pallas as pl
from jax.experimental.pallas import tpu as pltpu
from jax.experimental.pallas import tpu_sc as plsc

_N_IN = 16384
_N_OUT = 16384
_B = 64
_NC = 2
_NS = 16
_NW = _NC * _NS
_G = 128


_NBG = 4


def _sc_body(xT_ref, cols_ref, rows_ref, w_ref, out_ref,
             acc, cols_l, rows_l, w_l,
             buf0, buf1, buf2, buf3,
             gsem0, gsem1, gsem2, gsem3, ssem):
    c = lax.axis_index("c")
    s = lax.axis_index("s")
    wid = s * _NC + c
    nchunk = cols_l.shape[0]
    bufs = (buf0, buf1, buf2, buf3)
    gsems = (gsem0, gsem1, gsem2, gsem3)

    pltpu.sync_copy(cols_ref.at[wid], cols_l)
    pltpu.sync_copy(rows_ref.at[wid], rows_l)
    pltpu.sync_copy(w_ref.at[wid], w_l)

    zero16 = jnp.zeros((16,), jnp.float32)

    def _zb(i, carry):
        for r in bufs:
            for j in range(_B // 16):
                r[i, pl.ds(j * 16, 16)] = zero16
        return carry

    lax.fori_loop(0, _G, _zb, 0)
    rows_per = _N_OUT // _NS
    for k in range(rows_per // _G):
        pltpu.sync_copy(bufs[k % _NBG], acc.at[pl.ds(s * rows_per + k * _G, _G)])
    plsc.subcore_barrier()
    for b in range(_NBG):
        pltpu.async_copy(xT_ref.at[cols_l.at[b]], bufs[b], gsems[b])

    def _turn(i, b):
        pltpu.make_async_copy(xT_ref.at[cols_l.at[0]], bufs[b], gsems[b]).wait()
        base_i = jnp.full((16,), i * _G, jnp.int32)

        @plsc.parallel_loop(0, _G // 16, unroll=2)
        def _scale(q):
            lane0 = base_i + q * 16
            for l in range(16):
                wb = plsc.load_gather(w_l, [lane0 + l])
                g = q * 16 + l
                for j in range(_B // 16):
                    sl = pl.ds(j * 16, 16)
                    bufs[b][g, sl] = bufs[b][g, sl] * wb
        pltpu.async_copy(bufs[b], acc.at[rows_l.at[i]], ssem, add=True)
        pltpu.make_async_copy(bufs[b], acc.at[rows_l.at[0]], ssem).wait()

        @pl.when(i + _NBG < nchunk)
        def _refill():
            pltpu.async_copy(xT_ref.at[cols_l.at[i + _NBG]], bufs[b], gsems[b])

    def _chunk(k, carry):
        for b in range(_NBG):
            _turn(k * _NBG + b, b)
        return carry

    lax.fori_loop(0, nchunk // _NBG, _chunk, 0)
    plsc.subcore_barrier()

    pltpu.sync_copy(acc.at[pl.ds(s * rows_per, rows_per)],
                    out_ref.at[c, pl.ds(s * rows_per, rows_per)])


def _combine_body(p_ref, b_ref, o_ref):
    t = p_ref[0] + p_ref[1]
    o_ref[...] = t.T + b_ref[...]


def kernel(x, sparse_weight, bias, rows, cols):
    nnz = sparse_weight.shape[0]
    nchunk = -(-nnz // (_NW * _G))
    nchunk = -(-nchunk // _NBG) * _NBG
    total = _NW * nchunk * _G
    pad = total - nnz

    cols_p = jnp.concatenate(
        [cols, jnp.zeros((pad,), jnp.int32)]).reshape(_NW, nchunk, _G)
    rows_p = jnp.concatenate(
        [rows, jnp.zeros((pad,), jnp.int32)]).reshape(_NW, nchunk, _G)
    w_p = jnp.concatenate(
        [sparse_weight, jnp.zeros((pad,), jnp.float32)]).reshape(
            _NW, nchunk * _G)
    xT = x.T

    mesh = plsc.VectorSubcoreMesh(
        core_axis_name="c", subcore_axis_name="s",
        num_cores=_NC, num_subcores=_NS)
    sck = pl.kernel(
        _sc_body,
        out_type=jax.ShapeDtypeStruct((_NC, _N_OUT, _B), jnp.float32),
        mesh=mesh,
        compiler_params=pltpu.CompilerParams(
            needs_layout_passes=False, use_tc_tiling_on_sc=False),
        scratch_types=[
            pltpu.VMEM_SHARED((_N_OUT, _B), jnp.float32),
            pltpu.VMEM((nchunk, _G), jnp.int32),
            pltpu.VMEM((nchunk, _G), jnp.int32),
            pltpu.VMEM((nchunk * _G,), jnp.float32),
        ] + [pltpu.VMEM((_G, _B), jnp.float32)] * _NBG
          + [pltpu.SemaphoreType.DMA] * (_NBG + 1),
    )
    partial = sck(xT, cols_p, rows_p, w_p)

    blk = 1024
    out = pl.pallas_call(
        _combine_body,
        grid=(_N_OUT // blk,),
        in_specs=[
            pl.BlockSpec((_NC, blk, _B), lambda i: (0, i, 0)),
            pl.BlockSpec((1, blk), lambda i: (0, i)),
        ],
        out_specs=pl.BlockSpec((_B, blk), lambda i: (0, i)),
        out_shape=jax.ShapeDtypeStruct((_B, _N_OUT), jnp.float32),
    )(partial, bias.reshape(1, _N_OUT))
    return out


# speedup vs baseline: 2.0065x; 2.0065x over previous
import functools

import jax
import jax.numpy as jnp
from jax import lax
from jax.experimental import pallas as pl
from jax.experimental.pallas import tpu as pltpu
from jax.experimental.pallas import tpu_sc as plsc

_N_IN = 16384
_N_OUT = 16384
_B = 64
_NC = 2
_NS = 16
_NW = _NC * _NS
_G = 128


_NBG = 3


def _sc_body(xT_ref, cols_ref, rows_ref, w_ref, out_ref,
             acc, cols_l, rows_l, w_l,
             buf0, buf1, buf2, obuf0,
             gsem0, gsem1, gsem2, ssem):
    c = lax.axis_index("c")
    s = lax.axis_index("s")
    wid = s * _NC + c
    nchunk = cols_l.shape[0]
    bufs = (buf0, buf1, buf2)
    obufs = (obuf0,)
    gsems = (gsem0, gsem1, gsem2)

    pltpu.sync_copy(cols_ref.at[wid], cols_l)
    pltpu.sync_copy(rows_ref.at[wid], rows_l)
    pltpu.sync_copy(w_ref.at[wid], w_l)

    zero16 = jnp.zeros((16,), jnp.float32)

    def _zb(i, carry):
        for r in bufs + obufs:
            for j in range(_B // 16):
                r[i, pl.ds(j * 16, 16)] = zero16
        return carry

    lax.fori_loop(0, _G, _zb, 0)
    rows_per = _N_OUT // _NS
    for k in range(rows_per // _G):
        pltpu.sync_copy(bufs[k % _NBG], acc.at[pl.ds(s * rows_per + k * _G, _G)])
    plsc.subcore_barrier()
    for b in range(_NBG):
        pltpu.async_copy(xT_ref.at[cols_l.at[b]], bufs[b], gsems[b])

    def _turn(i, b, ob):
        pltpu.make_async_copy(xT_ref.at[cols_l.at[0]], bufs[b], gsems[b]).wait()
        base_i = jnp.full((16,), i * _G, jnp.int32)

        @plsc.parallel_loop(0, _G // 16, unroll=2)
        def _scale(q):
            lane0 = base_i + q * 16
            for l in range(16):
                wb = plsc.load_gather(w_l, [lane0 + l])
                g = q * 16 + l
                for j in range(_B // 16):
                    sl = pl.ds(j * 16, 16)
                    obufs[ob][g, sl] = bufs[b][g, sl] * wb
        pltpu.async_copy(obufs[ob], acc.at[rows_l.at[i]], ssem, add=True)
        pltpu.make_async_copy(obufs[ob], acc.at[rows_l.at[0]], ssem).wait()

        @pl.when(i + _NBG < nchunk)
        def _refill():
            pltpu.async_copy(xT_ref.at[cols_l.at[i + _NBG]], bufs[b], gsems[b])

    def _chunk(k, carry):
        for b in range(_NBG):
            _turn(k * _NBG + b, b, 0)
        return carry

    lax.fori_loop(0, nchunk // _NBG, _chunk, 0)
    plsc.subcore_barrier()

    pltpu.sync_copy(acc.at[pl.ds(s * rows_per, rows_per)],
                    out_ref.at[c, pl.ds(s * rows_per, rows_per)])


def _combine_body(p_ref, b_ref, o_ref):
    t = p_ref[0] + p_ref[1]
    o_ref[...] = t.T + b_ref[...]


def kernel(x, sparse_weight, bias, rows, cols):
    nnz = sparse_weight.shape[0]
    nchunk = -(-nnz // (_NW * _G))
    nchunk = -(-nchunk // _NBG) * _NBG
    total = _NW * nchunk * _G
    pad = total - nnz

    cols_p = jnp.concatenate(
        [cols, jnp.zeros((pad,), jnp.int32)]).reshape(_NW, nchunk, _G)
    rows_p = jnp.concatenate(
        [rows, jnp.zeros((pad,), jnp.int32)]).reshape(_NW, nchunk, _G)
    w_p = jnp.concatenate(
        [sparse_weight, jnp.zeros((pad,), jnp.float32)]).reshape(
            _NW, nchunk * _G)
    xT = x.T

    mesh = plsc.VectorSubcoreMesh(
        core_axis_name="c", subcore_axis_name="s",
        num_cores=_NC, num_subcores=_NS)
    sck = pl.kernel(
        _sc_body,
        out_type=jax.ShapeDtypeStruct((_NC, _N_OUT, _B), jnp.float32),
        mesh=mesh,
        compiler_params=pltpu.CompilerParams(
            needs_layout_passes=False, use_tc_tiling_on_sc=False),
        scratch_types=[
            pltpu.VMEM_SHARED((_N_OUT, _B), jnp.float32),
            pltpu.VMEM((nchunk, _G), jnp.int32),
            pltpu.VMEM((nchunk, _G), jnp.int32),
            pltpu.VMEM((nchunk * _G,), jnp.float32),
        ] + [pltpu.VMEM((_G, _B), jnp.float32)] * (_NBG + 1)
          + [pltpu.SemaphoreType.DMA] * (_NBG + 1),
    )
    partial = sck(xT, cols_p, rows_p, w_p)

    blk = 1024
    out = pl.pallas_call(
        _combine_body,
        grid=(_N_OUT // blk,),
        in_specs=[
            pl.BlockSpec((_NC, blk, _B), lambda i: (0, i, 0)),
            pl.BlockSpec((1, blk), lambda i: (0, i)),
        ],
        out_specs=pl.BlockSpec((_B, blk), lambda i: (0, i)),
        out_shape=jax.ShapeDtypeStruct((_B, _N_OUT), jnp.float32),
    )(partial, bias.reshape(1, _N_OUT))
    return out
